# baseline (speedup 1.0000x reference)
import functools

import jax
import jax.numpy as jnp
from jax import lax
from jax.experimental import pallas as pl
from jax.experimental.pallas import tpu as pltpu
from jax.experimental.pallas import tpu_sc as plsc

N = 10000
NPAD = 10240
E = 320000
D = 128
H = 128
A_DIM = 8
MAX_ACT = 1.0

NC = 2
NS = 16
NW = NC * NS
C = 64
NCHUNKS = 168
PASSES = 6
CPP = NCHUNKS // PASSES
EPW = NCHUNKS * C
E_PAD = NW * EPW
ROWS_PER_TILE = NPAD // NS


DW = 128


@functools.partial(
    pl.kernel,
    mesh=plsc.VectorSubcoreMesh(core_axis_name="c", subcore_axis_name="s"),
    out_type=jax.ShapeDtypeStruct((NC, NPAD, DW), jnp.float32),
    scratch_types=[
        pltpu.VMEM((NCHUNKS, C), jnp.int32),
        pltpu.VMEM((C, DW), jnp.float32),
        pltpu.VMEM((C, DW), jnp.float32),
        pltpu.VMEM_SHARED((NPAD, DW), jnp.float32),
    ],
)
def _deg_sc(col_hbm, const_hbm, cnt_hbm, colidx, onesbuf, zbuf, acc):
    cid = lax.axis_index("c")
    sid = lax.axis_index("s")
    wid = sid * NC + cid
    pltpu.sync_copy(col_hbm.at[wid], colidx)
    pltpu.sync_copy(const_hbm.at[pl.ds(0, C)], onesbuf)
    pltpu.sync_copy(const_hbm.at[pl.ds(C, C)], zbuf)
    for k in range(ROWS_PER_TILE // C):
        pltpu.sync_copy(zbuf, acc.at[pl.ds(sid * ROWS_PER_TILE + k * C, C)])
    plsc.subcore_barrier()

    def step(j, _):
        pltpu.sync_copy(onesbuf, acc.at[colidx.at[j]], add=True)
        return 0

    lax.fori_loop(0, NCHUNKS, step, 0)
    plsc.subcore_barrier()
    for k in range(ROWS_PER_TILE // C):
        r = sid * ROWS_PER_TILE + k * C
        pltpu.sync_copy(acc.at[pl.ds(r, C)], zbuf)
        pltpu.sync_copy(zbuf, cnt_hbm.at[cid, pl.ds(r, C)])


def _tc_a_body(x_ref, w_ref, cnt_ref, g_ref, dis_ref):
    cnt = cnt_ref[...]
    deg = cnt[0, :, 0:1] + cnt[1, :, 0:1]
    dis_col = jnp.where(deg > 0.0,
                        lax.rsqrt(jnp.maximum(deg, 1e-12)), 0.0)
    h = jnp.dot(x_ref[...], w_ref[...], preferred_element_type=jnp.float32)
    g_ref[...] = h * dis_col
    dis_ref[...] = dis_col


def _tc_a(xp, w, cnt):
    return pl.pallas_call(
        _tc_a_body,
        out_shape=[
            jax.ShapeDtypeStruct((NPAD, H), jnp.float32),
            jax.ShapeDtypeStruct((NPAD, 1), jnp.float32),
        ],
    )(xp, w, cnt)


@functools.partial(
    pl.kernel,
    mesh=plsc.VectorSubcoreMesh(core_axis_name="c", subcore_axis_name="s"),
    out_type=jax.ShapeDtypeStruct((NC, NPAD, H), jnp.float32),
    scratch_types=[
        pltpu.VMEM((CPP, C), jnp.int32),
        pltpu.VMEM((CPP, C), jnp.int32),
        pltpu.VMEM((C, H), jnp.float32),
        pltpu.VMEM((C, H), jnp.float32),
        pltpu.VMEM((C, H), jnp.float32),
        pltpu.VMEM((C, H), jnp.float32),
        pltpu.VMEM_SHARED((NPAD, H), jnp.float32),
        pltpu.SemaphoreType.DMA,
        pltpu.SemaphoreType.DMA,
        pltpu.SemaphoreType.DMA,
        pltpu.SemaphoreType.DMA,
    ],
)
def _main_sc(g_hbm, row_hbm, col_hbm, z_hbm, p_hbm,
             rowidx, colidx, msg0, msg1, msg2, msg3, acc,
             sem0, sem1, sem2, sem3):
    cid = lax.axis_index("c")
    sid = lax.axis_index("s")
    wid = sid * NC + cid
    pltpu.sync_copy(z_hbm, msg0)
    for k in range(ROWS_PER_TILE // C):
        pltpu.sync_copy(msg0, acc.at[pl.ds(sid * ROWS_PER_TILE + k * C, C)])
    plsc.subcore_barrier()

    def step(jj, _):
        base = jj * 4

        def quarter(r, msg, sem):
            j = base + r
            pltpu.make_async_copy(g_hbm.at[rowidx.at[j]], msg, sem).wait()
            pltpu.sync_copy(msg, acc.at[colidx.at[j]], add=True)

            @pl.when(jj < CPP // 4 - 1)
            def _prefetch():
                pltpu.async_copy(g_hbm.at[rowidx.at[j + 4]], msg, sem)

        quarter(0, msg0, sem0)
        quarter(1, msg1, sem1)
        quarter(2, msg2, sem2)
        quarter(3, msg3, sem3)
        return 0

    for p in range(PASSES):
        pltpu.sync_copy(row_hbm.at[wid, p], rowidx)
        pltpu.sync_copy(col_hbm.at[wid, p], colidx)
        pltpu.async_copy(g_hbm.at[rowidx.at[0]], msg0, sem0)
        pltpu.async_copy(g_hbm.at[rowidx.at[1]], msg1, sem1)
        pltpu.async_copy(g_hbm.at[rowidx.at[2]], msg2, sem2)
        pltpu.async_copy(g_hbm.at[rowidx.at[3]], msg3, sem3)
        lax.fori_loop(0, CPP // 4, step, 0)
    plsc.subcore_barrier()
    for k in range(ROWS_PER_TILE // C):
        r = sid * ROWS_PER_TILE + k * C
        pltpu.sync_copy(acc.at[pl.ds(r, C)], msg0)
        pltpu.sync_copy(msg0, p_hbm.at[cid, pl.ds(r, C)])


def _tc_b_body(p_ref, dis_ref, bconv_ref, gamma_ref, beta_ref,
               w2_ref, b2_ref, w3_ref, b3_ref, out_ref):
    p = p_ref[...]
    s = p[0] + p[1]
    out = dis_ref[...] * s + bconv_ref[...]
    z = jnp.maximum(out, 0.0)
    mu = jnp.mean(z, axis=1, keepdims=True)
    zc = z - mu
    var = jnp.mean(zc * zc, axis=1, keepdims=True)
    y = zc * lax.rsqrt(var + 1e-5) * gamma_ref[...] + beta_ref[...]
    rowids = lax.broadcasted_iota(jnp.int32, (NPAD, 1), 0)
    y = jnp.where(rowids < N, y, 0.0)
    pooled = jnp.sum(y, axis=0, keepdims=True)
    a = jnp.maximum(
        jnp.dot(pooled, w2_ref[...], preferred_element_type=jnp.float32)
        + b2_ref[...], 0.0)
    out_ref[...] = MAX_ACT * jnp.tanh(
        jnp.dot(a, w3_ref[...], preferred_element_type=jnp.float32)
        + b3_ref[...])


def _tc_b(p, dis, bconv, gamma, beta, w2, b2, w3, b3):
    return pl.pallas_call(
        _tc_b_body,
        out_shape=jax.ShapeDtypeStruct((1, A_DIM), jnp.float32),
    )(p, dis, bconv, gamma, beta, w2, b2, w3, b3)


def kernel(x, edge_index, W_conv, b_conv, gamma, beta, W2, b2, W3, b3):
    xp = jnp.pad(x, ((0, NPAD - N), (0, 0)))
    loop = jnp.arange(N, dtype=jnp.int32)
    npad_edges = E_PAD - (E + N)
    pad_idx = N + (jnp.arange(npad_edges, dtype=jnp.int32) % (NPAD - N))
    row = jnp.concatenate([edge_index[0], loop, pad_idx])
    col = jnp.concatenate([edge_index[1], loop, pad_idx])
    row4 = row.reshape(NW, PASSES, CPP, C)
    col4 = col.reshape(NW, PASSES, CPP, C)
    col3 = col.reshape(NW, NCHUNKS, C)

    onehot = jnp.zeros((2 * C, DW), jnp.float32).at[:C, 0].set(1.0)
    cnt = _deg_sc(col3, onehot)
    g, dis = _tc_a(xp, W_conv, cnt)
    zeros = jnp.zeros((C, H), jnp.float32)
    p = _main_sc(g, row4, col4, zeros)
    return _tc_b(p, dis, b_conv.reshape(1, H), gamma.reshape(1, H),
                 beta.reshape(1, H), W2, b2.reshape(1, H), W3,
                 b3.reshape(1, A_DIM))


# speedup vs baseline: 1.0073x; 1.0073x over previous
import functools

import jax
import jax.numpy as jnp
from jax import lax
from jax.experimental import pallas as pl
from jax.experimental.pallas import tpu as pltpu
from jax.experimental.pallas import tpu_sc as plsc

N = 10000
NPAD = 10240
E = 320000
D = 128
H = 128
A_DIM = 8
MAX_ACT = 1.0

NC = 2
NS = 16
NW = NC * NS
C = 128
NCHUNKS = 84
PASSES = 2
CPP = NCHUNKS // PASSES
EPW = NCHUNKS * C
E_PAD = NW * EPW
ROWS_PER_TILE = NPAD // NS


DW = 128


@functools.partial(
    pl.kernel,
    mesh=plsc.VectorSubcoreMesh(core_axis_name="c", subcore_axis_name="s"),
    out_type=jax.ShapeDtypeStruct((NC, NPAD, DW), jnp.float32),
    scratch_types=[
        pltpu.VMEM((NCHUNKS, C), jnp.int32),
        pltpu.VMEM((C, DW), jnp.float32),
        pltpu.VMEM((C, DW), jnp.float32),
        pltpu.VMEM_SHARED((NPAD, DW), jnp.float32),
    ],
)
def _deg_sc(col_hbm, const_hbm, cnt_hbm, colidx, onesbuf, zbuf, acc):
    cid = lax.axis_index("c")
    sid = lax.axis_index("s")
    wid = sid * NC + cid
    pltpu.sync_copy(col_hbm.at[wid], colidx)
    pltpu.sync_copy(const_hbm.at[pl.ds(0, C)], onesbuf)
    pltpu.sync_copy(const_hbm.at[pl.ds(C, C)], zbuf)
    for k in range(ROWS_PER_TILE // C):
        pltpu.sync_copy(zbuf, acc.at[pl.ds(sid * ROWS_PER_TILE + k * C, C)])
    plsc.subcore_barrier()

    def step(j, _):
        pltpu.sync_copy(onesbuf, acc.at[colidx.at[j]], add=True)
        return 0

    lax.fori_loop(0, NCHUNKS, step, 0)
    plsc.subcore_barrier()
    for k in range(ROWS_PER_TILE // C):
        r = sid * ROWS_PER_TILE + k * C
        pltpu.sync_copy(acc.at[pl.ds(r, C)], zbuf)
        pltpu.sync_copy(zbuf, cnt_hbm.at[cid, pl.ds(r, C)])


def _tc_a_body(x_ref, w_ref, cnt_ref, g_ref, dis_ref):
    cnt = cnt_ref[...]
    deg = cnt[0, :, 0:1] + cnt[1, :, 0:1]
    dis_col = jnp.where(deg > 0.0,
                        lax.rsqrt(jnp.maximum(deg, 1e-12)), 0.0)
    h = jnp.dot(x_ref[...], w_ref[...], preferred_element_type=jnp.float32)
    g_ref[...] = h * dis_col
    dis_ref[...] = dis_col


def _tc_a(xp, w, cnt):
    return pl.pallas_call(
        _tc_a_body,
        out_shape=[
            jax.ShapeDtypeStruct((NPAD, H), jnp.float32),
            jax.ShapeDtypeStruct((NPAD, 1), jnp.float32),
        ],
    )(xp, w, cnt)


@functools.partial(
    pl.kernel,
    mesh=plsc.VectorSubcoreMesh(core_axis_name="c", subcore_axis_name="s"),
    out_type=jax.ShapeDtypeStruct((NC, NPAD, H), jnp.float32),
    scratch_types=[
        pltpu.VMEM((CPP, C), jnp.int32),
        pltpu.VMEM((CPP, C), jnp.int32),
        pltpu.VMEM((C, H), jnp.float32),
        pltpu.VMEM((C, H), jnp.float32),
        pltpu.VMEM_SHARED((NPAD, H), jnp.float32),
        pltpu.SemaphoreType.DMA,
        pltpu.SemaphoreType.DMA,
    ],
)
def _main_sc(g_hbm, row_hbm, col_hbm, z_hbm, p_hbm,
             rowidx, colidx, msg0, msg1, acc, sem0, sem1):
    cid = lax.axis_index("c")
    sid = lax.axis_index("s")
    wid = sid * NC + cid
    pltpu.sync_copy(z_hbm, msg0)
    for k in range(ROWS_PER_TILE // C):
        pltpu.sync_copy(msg0, acc.at[pl.ds(sid * ROWS_PER_TILE + k * C, C)])
    plsc.subcore_barrier()

    def step(jj, _):
        j0 = jj * 2
        j1 = j0 + 1
        pltpu.async_copy(g_hbm.at[rowidx.at[j1]], msg1, sem1)
        pltpu.make_async_copy(g_hbm.at[rowidx.at[j0]], msg0, sem0).wait()
        pltpu.sync_copy(msg0, acc.at[colidx.at[j0]], add=True)

        @pl.when(jj < CPP // 2 - 1)
        def _prefetch():
            pltpu.async_copy(g_hbm.at[rowidx.at[j0 + 2]], msg0, sem0)

        pltpu.make_async_copy(g_hbm.at[rowidx.at[j1]], msg1, sem1).wait()
        pltpu.sync_copy(msg1, acc.at[colidx.at[j1]], add=True)
        return 0

    for p in range(PASSES):
        pltpu.sync_copy(row_hbm.at[wid, p], rowidx)
        pltpu.sync_copy(col_hbm.at[wid, p], colidx)
        pltpu.async_copy(g_hbm.at[rowidx.at[0]], msg0, sem0)
        lax.fori_loop(0, CPP // 2, step, 0)
    plsc.subcore_barrier()
    for k in range(ROWS_PER_TILE // C):
        r = sid * ROWS_PER_TILE + k * C
        pltpu.sync_copy(acc.at[pl.ds(r, C)], msg0)
        pltpu.sync_copy(msg0, p_hbm.at[cid, pl.ds(r, C)])


def _tc_b_body(p_ref, dis_ref, bconv_ref, gamma_ref, beta_ref,
               w2_ref, b2_ref, w3_ref, b3_ref, out_ref):
    p = p_ref[...]
    s = p[0] + p[1]
    out = dis_ref[...] * s + bconv_ref[...]
    z = jnp.maximum(out, 0.0)
    mu = jnp.mean(z, axis=1, keepdims=True)
    zc = z - mu
    var = jnp.mean(zc * zc, axis=1, keepdims=True)
    y = zc * lax.rsqrt(var + 1e-5) * gamma_ref[...] + beta_ref[...]
    rowids = lax.broadcasted_iota(jnp.int32, (NPAD, 1), 0)
    y = jnp.where(rowids < N, y, 0.0)
    pooled = jnp.sum(y, axis=0, keepdims=True)
    a = jnp.maximum(
        jnp.dot(pooled, w2_ref[...], preferred_element_type=jnp.float32)
        + b2_ref[...], 0.0)
    out_ref[...] = MAX_ACT * jnp.tanh(
        jnp.dot(a, w3_ref[...], preferred_element_type=jnp.float32)
        + b3_ref[...])


def _tc_b(p, dis, bconv, gamma, beta, w2, b2, w3, b3):
    return pl.pallas_call(
        _tc_b_body,
        out_shape=jax.ShapeDtypeStruct((1, A_DIM), jnp.float32),
    )(p, dis, bconv, gamma, beta, w2, b2, w3, b3)


def kernel(x, edge_index, W_conv, b_conv, gamma, beta, W2, b2, W3, b3):
    xp = jnp.pad(x, ((0, NPAD - N), (0, 0)))
    loop = jnp.arange(N, dtype=jnp.int32)
    npad_edges = E_PAD - (E + N)
    pad_idx = N + (jnp.arange(npad_edges, dtype=jnp.int32) % (NPAD - N))
    row = jnp.concatenate([edge_index[0], loop, pad_idx])
    col = jnp.concatenate([edge_index[1], loop, pad_idx])
    row4 = row.reshape(NW, PASSES, CPP, C)
    col4 = col.reshape(NW, PASSES, CPP, C)
    col3 = col.reshape(NW, NCHUNKS, C)

    onehot = jnp.zeros((2 * C, DW), jnp.float32).at[:C, 0].set(1.0)
    cnt = _deg_sc(col3, onehot)
    g, dis = _tc_a(xp, W_conv, cnt)
    zeros = jnp.zeros((C, H), jnp.float32)
    p = _main_sc(g, row4, col4, zeros)
    return _tc_b(p, dis, b_conv.reshape(1, H), gamma.reshape(1, H),
                 beta.reshape(1, H), W2, b2.reshape(1, H), W3,
                 b3.reshape(1, A_DIM))
